# Initial kernel scaffold; baseline (speedup 1.0000x reference)
import jax
import jax.numpy as jnp
from jax.experimental import pallas as pl


def kernel(xyz, new_xyz, features):
    raise NotImplementedError("write your pallas kernel here")


# speedup vs baseline: 2.5716x; 2.5716x over previous
import functools

import jax
import jax.numpy as jnp
from jax.experimental import pallas as pl

RADIUS2 = 0.25 * 0.25
NSAMPLE = 32


def _qag_block(xyz_ref, xyzT_ref, qT_ref, feat_ref, out_ref, *, C, N, QT):
    p2 = xyz_ref[0]
    pT = xyzT_ref[0]
    qT = qT_ref[0]
    f = feat_ref[0]

    d2 = None
    for d in range(3):
        dd = p2[:, d : d + 1] - qT[d : d + 1, :]
        sq = dd * dd
        d2 = sq if d2 is None else d2 + sq
    within = d2 < RADIUS2
    wf = within.astype(jnp.float32)
    CH = 128
    r_io = jax.lax.broadcasted_iota(jnp.int32, (CH, CH), 0)
    c_io = jax.lax.broadcasted_iota(jnp.int32, (CH, CH), 1)
    ltri = (r_io >= c_io).astype(jnp.float32)
    carry = jnp.zeros((1, QT), jnp.float32)
    rank_chunks = []
    for c0 in range(N // CH):
        chunk = wf[c0 * CH : (c0 + 1) * CH, :]
        cs = jax.lax.dot_general(ltri, chunk, (((1,), (0,)), ((), ())),
                                 preferred_element_type=jnp.float32)
        rank_chunks.append(cs + carry)
        carry = carry + cs[CH - 1 : CH, :]
    rank = jnp.concatenate(rank_chunks, axis=0)
    count = rank[N - 1 : N, :]
    rankm = jnp.where(within, rank, 0.0)
    iota0 = jax.lax.broadcasted_iota(jnp.int32, (N, QT), 0)
    ones_row = jnp.ones((1, N), jnp.float32)
    pe = jnp.concatenate([pT, ones_row], axis=0)

    g0f = None
    g0x = None
    empty_f = (count < 0.5).astype(jnp.float32)
    first_row = (iota0 == 0).astype(jnp.float32)
    for s in range(NSAMPLE):
        hsTf = (rankm == float(s + 1)).astype(jnp.float32)
        if s == 0:
            hsTf = hsTf + empty_f * first_row
        gf = jax.lax.dot_general(f, hsTf, (((1,), (0,)), ((), ())),
                                 preferred_element_type=jnp.float32)
        gx4 = jax.lax.dot_general(pe, hsTf, (((1,), (0,)), ((), ())),
                                  preferred_element_type=jnp.float32)
        gx = gx4[0:3]
        v = gx4[3:4]
        if s == 0:
            g0f, g0x = gf, gx
            sel_f, sel_x = gf, gx
        else:
            sel_f = v * gf + (1.0 - v) * g0f
            sel_x = v * gx + (1.0 - v) * g0x
        out_ref[0, 0:C, :, s] = sel_f
        out_ref[0, C : C + 3, :, s] = sel_x - qT


def kernel(xyz, new_xyz, features):
    B, N, _ = xyz.shape
    npoint = new_xyz.shape[1]
    C = features.shape[1]
    QT = 128 if npoint % 128 == 0 else npoint

    xyzT = jnp.transpose(xyz, (0, 2, 1))
    new_xyzT = jnp.transpose(new_xyz, (0, 2, 1))

    grid = (B, npoint // QT)
    body = functools.partial(_qag_block, C=C, N=N, QT=QT)
    out = pl.pallas_call(
        body,
        grid=grid,
        in_specs=[
            pl.BlockSpec((1, N, 3), lambda b, qt: (b, 0, 0)),
            pl.BlockSpec((1, 3, N), lambda b, qt: (b, 0, 0)),
            pl.BlockSpec((1, 3, QT), lambda b, qt: (b, 0, qt)),
            pl.BlockSpec((1, C, N), lambda b, qt: (b, 0, 0)),
        ],
        out_specs=pl.BlockSpec((1, C + 3, QT, NSAMPLE),
                               lambda b, qt: (b, 0, qt, 0)),
        out_shape=jax.ShapeDtypeStruct((B, C + 3, npoint, NSAMPLE),
                                       jnp.float32),
    )(xyz, xyzT, new_xyzT, features)
    return out


# speedup vs baseline: 13.0551x; 5.0767x over previous
import functools

import jax
import jax.numpy as jnp
from jax import lax
from jax.experimental import pallas as pl
from jax.experimental.pallas import tpu as pltpu
from jax.experimental.pallas import tpu_sc as plsc

RADIUS2 = 0.25 * 0.25
NSAMPLE = 32
WPQ = 256
QT = 128

NC, NS = 2, 16
NW = NC * NS


def _mask_pack_block(xyzT_ref, q_ref, pk_ref, *, N):
    pT = xyzT_ref[0]
    q = q_ref[0]
    d2 = None
    for d in range(3):
        dd = q[:, d : d + 1] - pT[d : d + 1, :]
        sq = dd * dd
        d2 = sq if d2 is None else d2 + sq
    wf = (d2 < RADIUS2).astype(jnp.bfloat16)
    i_io = lax.broadcasted_iota(jnp.int32, (N, N // 16), 0)
    g_io = lax.broadcasted_iota(jnp.int32, (N, N // 16), 1)
    pw = lax.shift_left(jnp.int32(1), i_io & 15)
    wt = jnp.where((i_io >> 4) == g_io, pw, 0).astype(jnp.bfloat16)
    pk = lax.dot_general(wf, wt, (((1,), (0,)), ((), ())),
                         preferred_element_type=jnp.float32)
    pk_ref[0] = pk.astype(jnp.int32)


def _mask_pack(xyz, new_xyz):
    B, N, _ = xyz.shape
    npoint = new_xyz.shape[1]
    xyzT = jnp.transpose(xyz, (0, 2, 1))
    return pl.pallas_call(
        functools.partial(_mask_pack_block, N=N),
        grid=(B, npoint // QT),
        in_specs=[
            pl.BlockSpec((1, 3, N), lambda b, qt: (b, 0, 0)),
            pl.BlockSpec((1, QT, 3), lambda b, qt: (b, qt, 0)),
        ],
        out_specs=pl.BlockSpec((1, QT, N // 16), lambda b, qt: (b, qt, 0)),
        out_shape=jax.ShapeDtypeStruct((B, npoint, N // 16), jnp.int32),
    )(xyzT, new_xyz)


def _select_kernel_body(pk_hbm, idx_hbm, pk_v, out_v, *, nq_total):
    wid = lax.axis_index("s") * NC + lax.axis_index("c")
    qpw = nq_total // NW
    nch = qpw // 64
    lanes = lax.broadcasted_iota(jnp.int32, (16,), 0)

    def chunk_body(ch, _):
        q0 = wid * qpw + ch * 64
        pltpu.sync_copy(pk_hbm.at[pl.ds(q0 * WPQ, 64 * WPQ)], pk_v)

        def group_body(g, _):
            qlane = g * 16 + lanes
            zero = jnp.zeros((16,), jnp.int32)

            def step(_, carry):
                wptr, wcur, cur, cnt, first = carry
                active = cnt < 32
                adv = (cur == 0) & (wptr < WPQ) & active
                wclamp = jnp.minimum(wptr, WPQ - 1)
                neww = plsc.load_gather(pk_v, [qlane * WPQ + wclamp])
                wcur = jnp.where(adv, wptr, wcur)
                cur = jnp.where(adv, neww, cur)
                wptr = wptr + adv.astype(jnp.int32)
                emit = active & (cur != 0)
                low = cur & (0 - cur)
                fbits = lax.bitcast_convert_type(low.astype(jnp.float32),
                                                jnp.int32)
                pos = lax.shift_right_logical(fbits, 23) - 127
                pidx = (wcur << 4) + pos
                first = jnp.where(emit & (cnt == 0), pidx, first)
                slot = (qlane << 5) + jnp.minimum(cnt, 31)
                plsc.store_scatter(out_v, [slot], pidx, mask=emit)
                cur = jnp.where(emit, cur & (cur - 1), cur)
                cnt = cnt + emit.astype(jnp.int32)
                return wptr, wcur, cur, cnt, first

            init = (zero, zero, zero, zero, zero)
            _, _, _, cnt, first = lax.fori_loop(0, WPQ + NSAMPLE, step, init)

            def pad(s, _):
                slot = (qlane << 5) + s
                plsc.store_scatter(out_v, [slot], first, mask=s >= cnt)
                return 0

            lax.fori_loop(0, NSAMPLE, pad, 0)
            return 0

        lax.fori_loop(0, 4, group_body, 0)
        pltpu.sync_copy(out_v, idx_hbm.at[pl.ds(q0 * 32, 2048)])
        return 0

    lax.fori_loop(0, nch, chunk_body, 0)


def _select(pk):
    nq_total = pk.shape[0]
    pk = pk.reshape(nq_total * WPQ)
    mesh = plsc.VectorSubcoreMesh(core_axis_name="c", subcore_axis_name="s")
    return pl.kernel(
        functools.partial(_select_kernel_body, nq_total=nq_total),
        out_type=jax.ShapeDtypeStruct((nq_total * NSAMPLE,), jnp.int32),
        mesh=mesh,
        compiler_params=pltpu.CompilerParams(needs_layout_passes=False),
        scratch_types=[
            pltpu.VMEM((64 * WPQ,), jnp.int32),
            pltpu.VMEM((64 * NSAMPLE,), jnp.int32),
        ],
    )(pk)


def _gather_kernel_body(feat_hbm, idx_hbm, xyzT_hbm, nqT_hbm, out_hbm,
                        table_v, xt_v, ct_v, idx_v, stage_v,
                        *, B, C, N, npoint):
    wid = lax.axis_index("s") * NC + lax.axis_index("c")
    ncb = C // 8
    tpb = ncb + 1
    ntasks = B * tpb
    qcn = npoint // 256
    lanes = lax.broadcasted_iota(jnp.int32, (16,), 0)
    niter = 256 * NSAMPLE // 16

    def task_body(t, _):
        @pl.when(t < ntasks)
        def _():
            b = t // tpb
            k = t % tpb

            @pl.when(k < ncb)
            def _feature_task():
                for c in range(8):
                    pltpu.sync_copy(
                        feat_hbm.at[pl.ds((b * C + k * 8 + c) * N, N)],
                        table_v.at[pl.ds(c * N, N)])

                def qchunk(qc, _):
                    qbase = b * npoint + qc * 256
                    pltpu.sync_copy(
                        idx_hbm.at[pl.ds(qbase * NSAMPLE, 256 * NSAMPLE)],
                        idx_v)
                    for c in range(8):
                        csplat = jnp.full((16,), c * N, jnp.int32)

                        def gstep(i, _):
                            iv = idx_v[pl.ds(i * 16, 16)]
                            g = plsc.load_gather(table_v, [csplat + iv])
                            stage_v[pl.ds(i * 16, 16)] = g
                            return 0

                        lax.fori_loop(0, niter, gstep, 0)
                        pltpu.sync_copy(
                            stage_v,
                            out_hbm.at[pl.ds(
                                ((b * (C + 3) + k * 8 + c) * npoint
                                 + qc * 256) * NSAMPLE,
                                256 * NSAMPLE)])
                    return 0

                lax.fori_loop(0, qcn, qchunk, 0)

            @pl.when(k == ncb)
            def _xyz_task():
                for d in range(3):
                    pltpu.sync_copy(xyzT_hbm.at[pl.ds((b * 3 + d) * N, N)],
                                    xt_v.at[pl.ds(d * N, N)])
                    pltpu.sync_copy(
                        nqT_hbm.at[pl.ds((b * 3 + d) * npoint, npoint)],
                        ct_v.at[pl.ds(d * npoint, npoint)])

                def qchunk(qc, _):
                    qbase = b * npoint + qc * 256
                    pltpu.sync_copy(
                        idx_hbm.at[pl.ds(qbase * NSAMPLE, 256 * NSAMPLE)],
                        idx_v)
                    for d in range(3):
                        dsplat = jnp.full((16,), d * N, jnp.int32)
                        dqsplat = jnp.full((16,), d * npoint, jnp.int32)

                        def gstep(i, _):
                            iv = idx_v[pl.ds(i * 16, 16)]
                            g = plsc.load_gather(xt_v, [dsplat + iv])
                            qsplat = dqsplat + qc * 256 + lax.div(i, 2)
                            ctr = plsc.load_gather(ct_v, [qsplat])
                            stage_v[pl.ds(i * 16, 16)] = g - ctr
                            return 0

                        lax.fori_loop(0, niter, gstep, 0)
                        pltpu.sync_copy(
                            stage_v,
                            out_hbm.at[pl.ds(
                                ((b * (C + 3) + C + d) * npoint
                                 + qc * 256) * NSAMPLE,
                                256 * NSAMPLE)])
                    return 0

                lax.fori_loop(0, qcn, qchunk, 0)

        return 0

    ntask_rounds = -(-(B * tpb) // NW)
    lax.fori_loop(0, ntask_rounds, lambda r, _: task_body(wid + r * NW, _), 0)


def _gather(features, idx, xyzT, nqT, npoint):
    B, C, N = features.shape
    mesh = plsc.VectorSubcoreMesh(core_axis_name="c", subcore_axis_name="s")
    return pl.kernel(
        functools.partial(_gather_kernel_body, B=B, C=C, N=N,
                          npoint=npoint),
        out_type=jax.ShapeDtypeStruct((B * (C + 3) * npoint * NSAMPLE,),
                                      jnp.float32),
        mesh=mesh,
        compiler_params=pltpu.CompilerParams(needs_layout_passes=False),
        scratch_types=[
            pltpu.VMEM((8 * N,), jnp.float32),
            pltpu.VMEM((3 * N,), jnp.float32),
            pltpu.VMEM((3 * npoint,), jnp.float32),
            pltpu.VMEM((256 * NSAMPLE,), jnp.int32),
            pltpu.VMEM((256 * NSAMPLE,), jnp.float32),
        ],
    )(features.reshape(B * C * N), idx, xyzT.reshape(B * 3 * N),
      nqT.reshape(B * 3 * npoint))


def kernel(xyz, new_xyz, features):
    B, N, _ = xyz.shape
    npoint = new_xyz.shape[1]
    C = features.shape[1]

    pk = _mask_pack(xyz, new_xyz)
    idx = _select(pk.reshape(B * npoint, N // 16))
    xyzT = jnp.transpose(xyz, (0, 2, 1))
    nqT = jnp.transpose(new_xyz, (0, 2, 1))
    out = _gather(features, idx, xyzT, nqT, npoint)
    return out.reshape(B, C + 3, npoint, NSAMPLE)


# speedup vs baseline: 13.3651x; 1.0237x over previous
import functools

import jax
import jax.numpy as jnp
from jax import lax
from jax.experimental import pallas as pl
from jax.experimental.pallas import tpu as pltpu
from jax.experimental.pallas import tpu_sc as plsc

RADIUS2 = 0.25 * 0.25
NSAMPLE = 32
WPQ = 256
QT = 128

NC, NS = 2, 16
NW = NC * NS


def _mask_pack_block(xyzT_ref, q_ref, pk_ref, *, N):
    pT = xyzT_ref[0]
    q = q_ref[0]
    d2 = None
    for d in range(3):
        dd = q[:, d : d + 1] - pT[d : d + 1, :]
        sq = dd * dd
        d2 = sq if d2 is None else d2 + sq
    wf = (d2 < RADIUS2).astype(jnp.bfloat16)
    i_io = lax.broadcasted_iota(jnp.int32, (N, N // 16), 0)
    g_io = lax.broadcasted_iota(jnp.int32, (N, N // 16), 1)
    pw = lax.shift_left(jnp.int32(1), i_io & 15)
    wt = jnp.where((i_io >> 4) == g_io, pw, 0).astype(jnp.bfloat16)
    pk = lax.dot_general(wf, wt, (((1,), (0,)), ((), ())),
                         preferred_element_type=jnp.float32)
    pk_ref[0] = pk.astype(jnp.int32)


def _mask_pack(xyz, new_xyz):
    B, N, _ = xyz.shape
    npoint = new_xyz.shape[1]
    xyzT = jnp.transpose(xyz, (0, 2, 1))
    return pl.pallas_call(
        functools.partial(_mask_pack_block, N=N),
        grid=(B, npoint // QT),
        in_specs=[
            pl.BlockSpec((1, 3, N), lambda b, qt: (b, 0, 0)),
            pl.BlockSpec((1, QT, 3), lambda b, qt: (b, qt, 0)),
        ],
        out_specs=pl.BlockSpec((1, QT, N // 16), lambda b, qt: (b, qt, 0)),
        out_shape=jax.ShapeDtypeStruct((B, npoint, N // 16), jnp.int32),
    )(xyzT, new_xyz)


def _select_kernel_body(pk_hbm, idx_hbm, pk_v, out_v, *, nq_total):
    wid = lax.axis_index("s") * NC + lax.axis_index("c")
    qpw = nq_total // NW
    nch = qpw // 64
    lanes = lax.broadcasted_iota(jnp.int32, (16,), 0)

    def chunk_body(ch, _):
        q0 = wid * qpw + ch * 64
        pltpu.sync_copy(pk_hbm.at[pl.ds(q0 * WPQ, 64 * WPQ)], pk_v)

        def group_body(g, _):
            qlane = g * 16 + lanes
            zero = jnp.zeros((16,), jnp.int32)

            def step(_, carry):
                wptr, wcur, cur, cnt, first = carry
                active = cnt < 32
                adv = (cur == 0) & (wptr < WPQ) & active
                wclamp = jnp.minimum(wptr, WPQ - 1)
                neww = plsc.load_gather(pk_v, [qlane * WPQ + wclamp])
                wcur = jnp.where(adv, wptr, wcur)
                cur = jnp.where(adv, neww, cur)
                wptr = wptr + adv.astype(jnp.int32)
                emit = active & (cur != 0)
                low = cur & (0 - cur)
                fbits = lax.bitcast_convert_type(low.astype(jnp.float32),
                                                jnp.int32)
                pos = lax.shift_right_logical(fbits, 23) - 127
                pidx = (wcur << 4) + pos
                first = jnp.where(emit & (cnt == 0), pidx, first)
                slot = (qlane << 5) + jnp.minimum(cnt, 31)
                plsc.store_scatter(out_v, [slot], pidx, mask=emit)
                cur = jnp.where(emit, cur & (cur - 1), cur)
                cnt = cnt + emit.astype(jnp.int32)
                return wptr, wcur, cur, cnt, first

            init = (zero, zero, zero, zero, zero)
            _, _, _, cnt, first = lax.fori_loop(0, WPQ + NSAMPLE, step, init)

            def pad(s, _):
                slot = (qlane << 5) + s
                plsc.store_scatter(out_v, [slot], first, mask=s >= cnt)
                return 0

            lax.fori_loop(0, NSAMPLE, pad, 0)
            return 0

        lax.fori_loop(0, 4, group_body, 0)
        pltpu.sync_copy(out_v, idx_hbm.at[pl.ds(q0 * 32, 2048)])
        return 0

    lax.fori_loop(0, nch, chunk_body, 0)


def _select(pk):
    nq_total = pk.shape[0]
    pk = pk.reshape(nq_total * WPQ)
    mesh = plsc.VectorSubcoreMesh(core_axis_name="c", subcore_axis_name="s")
    return pl.kernel(
        functools.partial(_select_kernel_body, nq_total=nq_total),
        out_type=jax.ShapeDtypeStruct((nq_total * NSAMPLE,), jnp.int32),
        mesh=mesh,
        compiler_params=pltpu.CompilerParams(needs_layout_passes=False),
        scratch_types=[
            pltpu.VMEM((64 * WPQ,), jnp.int32),
            pltpu.VMEM((64 * NSAMPLE,), jnp.int32),
        ],
    )(pk)


def _gather_kernel_body(feat_hbm, idx_hbm, xyzT_hbm, nqT_hbm, out_hbm,
                        table_v, xt_v, ct_v, idx_v, stage0_v, stage1_v,
                        sem0, sem1, *, B, C, N, npoint):
    wid = lax.axis_index("s") * NC + lax.axis_index("c")
    ncb = C // 8
    tpb = ncb + 1
    ntasks = B * tpb
    QC = 128
    qcn = npoint // QC
    niter = QC * NSAMPLE // 16
    CHW = QC * NSAMPLE

    def out_off(b, ch, qc):
        return ((b * (C + 3) + ch) * npoint + qc * QC) * NSAMPLE

    def drain(stage_v, sem, nch):
        for c in range(nch):
            pltpu.make_async_copy(
                stage_v.at[pl.ds(c * CHW, CHW)],
                out_hbm.at[pl.ds(c * CHW, CHW)], sem).wait()

    def task_body(t, _):
        @pl.when(t < ntasks)
        def _():
            b = t // tpb
            k = t % tpb

            @pl.when(k < ncb)
            def _feature_task():
                for j in range(4):
                    pltpu.sync_copy(feat_hbm.at[(b * C + k * 8) // 2 + j],
                                    table_v.at[pl.ds(2 * j, 2)])

                def qchunk(qc, _):
                    qbase = b * npoint + qc * QC
                    pltpu.sync_copy(
                        idx_hbm.at[pl.ds(qbase * NSAMPLE, CHW)], idx_v)

                    def fill_and_send(stage_v, sem):
                        def gstep(i, _):
                            iv = idx_v[pl.ds(i * 16, 16)]
                            for c in range(8):
                                g = plsc.load_gather(
                                    table_v,
                                    [jnp.full((16,), c, jnp.int32), iv])
                                stage_v[pl.ds(c * CHW + i * 16, 16)] = g
                            return 0

                        lax.fori_loop(0, niter, gstep, 0)
                        for c in range(8):
                            pltpu.async_copy(
                                stage_v.at[pl.ds(c * CHW, CHW)],
                                out_hbm.at[pl.ds(out_off(b, k * 8 + c, qc),
                                                 CHW)], sem)

                    @pl.when(qc >= 2)
                    def _():
                        @pl.when(qc % 2 == 0)
                        def _():
                            drain(stage0_v, sem0, 8)

                        @pl.when(qc % 2 == 1)
                        def _():
                            drain(stage1_v, sem1, 8)

                    @pl.when(qc % 2 == 0)
                    def _():
                        fill_and_send(stage0_v, sem0)

                    @pl.when(qc % 2 == 1)
                    def _():
                        fill_and_send(stage1_v, sem1)

                    return 0

                lax.fori_loop(0, qcn, qchunk, 0)
                drain(stage0_v, sem0, 8)
                drain(stage1_v, sem1, 8)

            @pl.when(k == ncb)
            def _xyz_task():
                for d in range(3):
                    pltpu.sync_copy(xyzT_hbm.at[pl.ds((b * 3 + d) * N, N)],
                                    xt_v.at[pl.ds(d * N, N)])
                    pltpu.sync_copy(
                        nqT_hbm.at[pl.ds((b * 3 + d) * npoint, npoint)],
                        ct_v.at[pl.ds(d * npoint, npoint)])

                def qchunk(qc, _):
                    qbase = b * npoint + qc * QC
                    pltpu.sync_copy(
                        idx_hbm.at[pl.ds(qbase * NSAMPLE, CHW)], idx_v)

                    def fill_and_send(stage_v, sem):
                        def gstep(i, _):
                            iv = idx_v[pl.ds(i * 16, 16)]
                            qsplat = jnp.full((16,), qc * QC, jnp.int32) \
                                + lax.div(i, 2)
                            for d in range(3):
                                g = plsc.load_gather(
                                    xt_v, [jnp.full((16,), d * N, jnp.int32)
                                           + iv])
                                ctr = plsc.load_gather(
                                    ct_v, [jnp.full((16,), d * npoint,
                                                    jnp.int32) + qsplat])
                                stage_v[pl.ds(d * CHW + i * 16, 16)] = g - ctr
                            return 0

                        lax.fori_loop(0, niter, gstep, 0)
                        for d in range(3):
                            pltpu.async_copy(
                                stage_v.at[pl.ds(d * CHW, CHW)],
                                out_hbm.at[pl.ds(out_off(b, C + d, qc),
                                                 CHW)], sem)

                    @pl.when(qc >= 2)
                    def _():
                        @pl.when(qc % 2 == 0)
                        def _():
                            drain(stage0_v, sem0, 3)

                        @pl.when(qc % 2 == 1)
                        def _():
                            drain(stage1_v, sem1, 3)

                    @pl.when(qc % 2 == 0)
                    def _():
                        fill_and_send(stage0_v, sem0)

                    @pl.when(qc % 2 == 1)
                    def _():
                        fill_and_send(stage1_v, sem1)

                    return 0

                lax.fori_loop(0, qcn, qchunk, 0)
                drain(stage0_v, sem0, 3)
                drain(stage1_v, sem1, 3)

        return 0

    ntask_rounds = -(-(B * tpb) // NW)
    lax.fori_loop(0, ntask_rounds, lambda r, _: task_body(wid + r * NW, _), 0)


def _gather(features, idx, xyzT, nqT, npoint):
    B, C, N = features.shape
    mesh = plsc.VectorSubcoreMesh(core_axis_name="c", subcore_axis_name="s")
    QC = 128
    return pl.kernel(
        functools.partial(_gather_kernel_body, B=B, C=C, N=N,
                          npoint=npoint),
        out_type=jax.ShapeDtypeStruct((B * (C + 3) * npoint * NSAMPLE,),
                                      jnp.float32),
        mesh=mesh,
        compiler_params=pltpu.CompilerParams(needs_layout_passes=False),
        scratch_types=[
            pltpu.VMEM((8, N), jnp.float32),
            pltpu.VMEM((3 * N,), jnp.float32),
            pltpu.VMEM((3 * npoint,), jnp.float32),
            pltpu.VMEM((QC * NSAMPLE,), jnp.int32),
            pltpu.VMEM((8 * QC * NSAMPLE,), jnp.float32),
            pltpu.VMEM((8 * QC * NSAMPLE,), jnp.float32),
            pltpu.SemaphoreType.DMA,
            pltpu.SemaphoreType.DMA,
        ],
    )(features.reshape(B * C // 2, 2, N), idx, xyzT.reshape(B * 3 * N),
      nqT.reshape(B * 3 * npoint))


def kernel(xyz, new_xyz, features):
    B, N, _ = xyz.shape
    npoint = new_xyz.shape[1]
    C = features.shape[1]

    pk = _mask_pack(xyz, new_xyz)
    idx = _select(pk.reshape(B * npoint, N // 16))
    xyzT = jnp.transpose(xyz, (0, 2, 1))
    nqT = jnp.transpose(new_xyz, (0, 2, 1))
    out = _gather(features, idx, xyzT, nqT, npoint)
    return out.reshape(B, C + 3, npoint, NSAMPLE)


# speedup vs baseline: 14.3514x; 1.0738x over previous
import functools

import jax
import jax.numpy as jnp
from jax import lax
from jax.experimental import pallas as pl
from jax.experimental.pallas import tpu as pltpu
from jax.experimental.pallas import tpu_sc as plsc

RADIUS2 = 0.25 * 0.25
NSAMPLE = 32
WPQ = 256
QT = 128

NC, NS = 2, 16
NW = NC * NS


def _mask_pack_block(xyzT_ref, q_ref, pk_ref, *, N):
    pT = xyzT_ref[0]
    q = q_ref[0]
    d2 = None
    for d in range(3):
        dd = q[:, d : d + 1] - pT[d : d + 1, :]
        sq = dd * dd
        d2 = sq if d2 is None else d2 + sq
    wf = (d2 < RADIUS2).astype(jnp.bfloat16)
    i_io = lax.broadcasted_iota(jnp.int32, (N, N // 16), 0)
    g_io = lax.broadcasted_iota(jnp.int32, (N, N // 16), 1)
    pw = lax.shift_left(jnp.int32(1), i_io & 15)
    wt = jnp.where((i_io >> 4) == g_io, pw, 0).astype(jnp.bfloat16)
    pk = lax.dot_general(wf, wt, (((1,), (0,)), ((), ())),
                         preferred_element_type=jnp.float32)
    pk_ref[0] = pk.astype(jnp.int32)


def _mask_pack(xyz, new_xyz):
    B, N, _ = xyz.shape
    npoint = new_xyz.shape[1]
    xyzT = jnp.transpose(xyz, (0, 2, 1))
    return pl.pallas_call(
        functools.partial(_mask_pack_block, N=N),
        grid=(B, npoint // QT),
        in_specs=[
            pl.BlockSpec((1, 3, N), lambda b, qt: (b, 0, 0)),
            pl.BlockSpec((1, QT, 3), lambda b, qt: (b, qt, 0)),
        ],
        out_specs=pl.BlockSpec((1, QT, N // 16), lambda b, qt: (b, qt, 0)),
        out_shape=jax.ShapeDtypeStruct((B, npoint, N // 16), jnp.int32),
    )(xyzT, new_xyz)


def _select_kernel_body(pk_hbm, idx_hbm, pk_v, out_v, *, nq_total):
    wid = lax.axis_index("s") * NC + lax.axis_index("c")
    qpw = nq_total // NW
    nch = qpw // 64
    lanes = lax.broadcasted_iota(jnp.int32, (16,), 0)

    def chunk_body(ch, _):
        q0 = wid * qpw + ch * 64
        pltpu.sync_copy(pk_hbm.at[pl.ds(q0 * WPQ, 64 * WPQ)], pk_v)

        def group_body(g, _):
            qlane = g * 16 + lanes
            zero = jnp.zeros((16,), jnp.int32)

            def step(_, carry):
                wptr, wcur, cur, cnt, first = carry
                active = cnt < 32
                adv = (cur == 0) & (wptr < WPQ) & active
                wclamp = jnp.minimum(wptr, WPQ - 1)
                neww = plsc.load_gather(pk_v, [qlane * WPQ + wclamp])
                wcur = jnp.where(adv, wptr, wcur)
                cur = jnp.where(adv, neww, cur)
                wptr = wptr + adv.astype(jnp.int32)
                emit = active & (cur != 0)
                low = cur & (0 - cur)
                fbits = lax.bitcast_convert_type(low.astype(jnp.float32),
                                                jnp.int32)
                pos = lax.shift_right_logical(fbits, 23) - 127
                pidx = (wcur << 4) + pos
                first = jnp.where(emit & (cnt == 0), pidx, first)
                slot = (qlane << 5) + jnp.minimum(cnt, 31)
                plsc.store_scatter(out_v, [slot], pidx, mask=emit)
                cur = jnp.where(emit, cur & (cur - 1), cur)
                cnt = cnt + emit.astype(jnp.int32)
                return wptr, wcur, cur, cnt, first

            init = (zero, zero, zero, zero, zero)
            _, _, _, cnt, first = lax.fori_loop(0, WPQ + NSAMPLE, step, init)

            def pad(s, _):
                slot = (qlane << 5) + s
                plsc.store_scatter(out_v, [slot], first, mask=s >= cnt)
                return 0

            lax.fori_loop(0, NSAMPLE, pad, 0)
            return 0

        lax.fori_loop(0, 4, group_body, 0)
        pltpu.sync_copy(out_v, idx_hbm.at[pl.ds(q0 * 32, 2048)])
        return 0

    lax.fori_loop(0, nch, chunk_body, 0)


def _select(pk):
    nq_total = pk.shape[0]
    pk = pk.reshape(nq_total * WPQ)
    mesh = plsc.VectorSubcoreMesh(core_axis_name="c", subcore_axis_name="s")
    return pl.kernel(
        functools.partial(_select_kernel_body, nq_total=nq_total),
        out_type=jax.ShapeDtypeStruct((nq_total * NSAMPLE,), jnp.int32),
        mesh=mesh,
        compiler_params=pltpu.CompilerParams(needs_layout_passes=False),
        scratch_types=[
            pltpu.VMEM((64 * WPQ,), jnp.int32),
            pltpu.VMEM((64 * NSAMPLE,), jnp.int32),
        ],
    )(pk)


def _gather_kernel_body(feat_hbm, idx_hbm, xyzT_hbm, nqT_hbm, out_hbm,
                        table_v, xt_v, ct_v, idx_v, stage0_v, stage1_v,
                        sem0, sem1, *, B, C, N, npoint):
    wid = lax.axis_index("s") * NC + lax.axis_index("c")
    ncb = C // 8
    tpb = ncb + 1
    ntasks = B * tpb
    QC = 128
    qcn = npoint // QC
    niter = QC * NSAMPLE // 16
    CHW = QC * NSAMPLE
    lanes = lax.broadcasted_iota(jnp.int32, (16,), 0)

    def drain(stage_v, sem, nch):
        for c in range(nch):
            pltpu.make_async_copy(
                stage_v.at[pl.ds(c * NSAMPLE, NSAMPLE), :],
                out_hbm.at[0, c, :, pl.ds(0, QC)], sem).wait()

    def task_body(t, _):
        @pl.when(t < ntasks)
        def _():
            b = t // tpb
            k = t % tpb

            @pl.when(k < ncb)
            def _feature_task():
                pltpu.sync_copy(feat_hbm.at[(b * C + k * 8) // 8],
                                table_v)

                def qchunk(qc, _):
                    qbase = b * npoint + qc * QC
                    pltpu.sync_copy(
                        idx_hbm.at[pl.ds(qbase * NSAMPLE, CHW)], idx_v)

                    def fill_and_send(stage_v, sem):
                        def gstep(i, _):
                            iv = idx_v[pl.ds(i * 16, 16)]
                            q_in = lax.shift_right_logical(i, 1)
                            s_vec = (i % 2) * 16 + lanes
                            qsp = jnp.full((16,), q_in, jnp.int32)
                            for c in range(8):
                                g = plsc.load_gather(
                                    table_v,
                                    [jnp.full((16,), c, jnp.int32), iv])
                                plsc.store_scatter(
                                    stage_v, [c * NSAMPLE + s_vec, qsp], g)
                            return 0

                        lax.fori_loop(0, niter, gstep, 0)
                        for c in range(8):
                            pltpu.async_copy(
                                stage_v.at[pl.ds(c * NSAMPLE, NSAMPLE), :],
                                out_hbm.at[b, k * 8 + c, :,
                                           pl.ds(qc * QC, QC)], sem)

                    @pl.when(qc >= 2)
                    def _():
                        @pl.when(qc % 2 == 0)
                        def _():
                            drain(stage0_v, sem0, 8)

                        @pl.when(qc % 2 == 1)
                        def _():
                            drain(stage1_v, sem1, 8)

                    @pl.when(qc % 2 == 0)
                    def _():
                        fill_and_send(stage0_v, sem0)

                    @pl.when(qc % 2 == 1)
                    def _():
                        fill_and_send(stage1_v, sem1)

                    return 0

                lax.fori_loop(0, qcn, qchunk, 0)
                drain(stage0_v, sem0, 8)
                drain(stage1_v, sem1, 8)

            @pl.when(k == ncb)
            def _xyz_task():
                for d in range(3):
                    pltpu.sync_copy(xyzT_hbm.at[pl.ds((b * 3 + d) * N, N)],
                                    xt_v.at[pl.ds(d * N, N)])
                    pltpu.sync_copy(
                        nqT_hbm.at[pl.ds((b * 3 + d) * npoint, npoint)],
                        ct_v.at[pl.ds(d * npoint, npoint)])

                def qchunk(qc, _):
                    qbase = b * npoint + qc * QC
                    pltpu.sync_copy(
                        idx_hbm.at[pl.ds(qbase * NSAMPLE, CHW)], idx_v)

                    def fill_and_send(stage_v, sem):
                        def gstep(i, _):
                            iv = idx_v[pl.ds(i * 16, 16)]
                            q_in = lax.shift_right_logical(i, 1)
                            s_vec = (i % 2) * 16 + lanes
                            qsp = jnp.full((16,), q_in, jnp.int32)
                            ctrq = jnp.full((16,), qc * QC, jnp.int32) + q_in
                            for d in range(3):
                                g = plsc.load_gather(
                                    xt_v, [jnp.full((16,), d * N, jnp.int32)
                                           + iv])
                                ctr = plsc.load_gather(
                                    ct_v, [jnp.full((16,), d * npoint,
                                                    jnp.int32) + ctrq])
                                plsc.store_scatter(
                                    stage_v, [d * NSAMPLE + s_vec, qsp],
                                    g - ctr)
                            return 0

                        lax.fori_loop(0, niter, gstep, 0)
                        for d in range(3):
                            pltpu.async_copy(
                                stage_v.at[pl.ds(d * NSAMPLE, NSAMPLE), :],
                                out_hbm.at[b, C + d, :,
                                           pl.ds(qc * QC, QC)], sem)

                    @pl.when(qc >= 2)
                    def _():
                        @pl.when(qc % 2 == 0)
                        def _():
                            drain(stage0_v, sem0, 3)

                        @pl.when(qc % 2 == 1)
                        def _():
                            drain(stage1_v, sem1, 3)

                    @pl.when(qc % 2 == 0)
                    def _():
                        fill_and_send(stage0_v, sem0)

                    @pl.when(qc % 2 == 1)
                    def _():
                        fill_and_send(stage1_v, sem1)

                    return 0

                lax.fori_loop(0, qcn, qchunk, 0)
                drain(stage0_v, sem0, 3)
                drain(stage1_v, sem1, 3)

        return 0

    ntask_rounds = -(-(B * tpb) // NW)
    lax.fori_loop(0, ntask_rounds, lambda r, _: task_body(wid + r * NW, _), 0)


def _gather(features, idx, xyzT, nqT, npoint):
    B, C, N = features.shape
    mesh = plsc.VectorSubcoreMesh(core_axis_name="c", subcore_axis_name="s")
    QC = 128
    return pl.kernel(
        functools.partial(_gather_kernel_body, B=B, C=C, N=N,
                          npoint=npoint),
        out_type=jax.ShapeDtypeStruct((B, C + 3, NSAMPLE, npoint),
                                      jnp.float32),
        mesh=mesh,
        compiler_params=pltpu.CompilerParams(needs_layout_passes=False),
        scratch_types=[
            pltpu.VMEM((8, N), jnp.float32),
            pltpu.VMEM((3 * N,), jnp.float32),
            pltpu.VMEM((3 * npoint,), jnp.float32),
            pltpu.VMEM((QC * NSAMPLE,), jnp.int32),
            pltpu.VMEM((8 * NSAMPLE, QC), jnp.float32),
            pltpu.VMEM((8 * NSAMPLE, QC), jnp.float32),
            pltpu.SemaphoreType.DMA,
            pltpu.SemaphoreType.DMA,
        ],
    )(features.reshape(B * C // 8, 8, N), idx, xyzT.reshape(B * 3 * N),
      nqT.reshape(B * 3 * npoint))


def kernel(xyz, new_xyz, features):
    B, N, _ = xyz.shape
    npoint = new_xyz.shape[1]
    C = features.shape[1]

    pk = _mask_pack(xyz, new_xyz)
    idx = _select(pk.reshape(B * npoint, N // 16))
    xyzT = jnp.transpose(xyz, (0, 2, 1))
    nqT = jnp.transpose(new_xyz, (0, 2, 1))
    out_sq = _gather(features, idx, xyzT, nqT, npoint)
    return jnp.transpose(out_sq, (0, 1, 3, 2))


# speedup vs baseline: 26.1042x; 1.8189x over previous
import functools

import jax
import jax.numpy as jnp
from jax import lax
from jax.experimental import pallas as pl
from jax.experimental.pallas import tpu as pltpu
from jax.experimental.pallas import tpu_sc as plsc

RADIUS2 = 0.25 * 0.25
NSAMPLE = 32
WPQ = 256
QT = 128

NC, NS = 2, 16
NW = NC * NS


def _mask_pack_block(xyzT_ref, q_ref, pk_ref, *, N):
    pT = xyzT_ref[0]
    q = q_ref[0]
    d2 = None
    for d in range(3):
        dd = q[:, d : d + 1] - pT[d : d + 1, :]
        sq = dd * dd
        d2 = sq if d2 is None else d2 + sq
    wf = (d2 < RADIUS2).astype(jnp.bfloat16)
    i_io = lax.broadcasted_iota(jnp.int32, (N, N // 16), 0)
    g_io = lax.broadcasted_iota(jnp.int32, (N, N // 16), 1)
    pw = lax.shift_left(jnp.int32(1), i_io & 15)
    wt = jnp.where((i_io >> 4) == g_io, pw, 0).astype(jnp.bfloat16)
    pk = lax.dot_general(wf, wt, (((1,), (0,)), ((), ())),
                         preferred_element_type=jnp.float32)
    pk_ref[0] = pk.astype(jnp.int32)


def _mask_pack(xyz, new_xyz):
    B, N, _ = xyz.shape
    npoint = new_xyz.shape[1]
    xyzT = jnp.transpose(xyz, (0, 2, 1))
    return pl.pallas_call(
        functools.partial(_mask_pack_block, N=N),
        grid=(B, npoint // QT),
        in_specs=[
            pl.BlockSpec((1, 3, N), lambda b, qt: (b, 0, 0)),
            pl.BlockSpec((1, QT, 3), lambda b, qt: (b, qt, 0)),
        ],
        out_specs=pl.BlockSpec((1, QT, N // 16), lambda b, qt: (b, qt, 0)),
        out_shape=jax.ShapeDtypeStruct((B, npoint, N // 16), jnp.int32),
    )(xyzT, new_xyz)


def _select_kernel_body(pk_hbm, idx_hbm, pk_v, out_v, *, nq_total):
    wid = lax.axis_index("s") * NC + lax.axis_index("c")
    qpw = nq_total // NW
    nch = qpw // 128
    lanes = lax.broadcasted_iota(jnp.int32, (16,), 0)

    def chunk_body(ch, _):
        q0 = wid * qpw + ch * 128
        pltpu.sync_copy(pk_hbm.at[pl.ds(q0 * WPQ, 128 * WPQ)], pk_v)

        def group_body(g, _):
            qlane = g * 16 + lanes
            zero = jnp.zeros((16,), jnp.int32)

            def step(_, carry):
                wptr, wcur, cur, cnt, first = carry
                active = cnt < 32
                adv = (cur == 0) & (wptr < WPQ) & active
                wclamp = jnp.minimum(wptr, WPQ - 1)
                neww = plsc.load_gather(pk_v, [qlane * WPQ + wclamp])
                wcur = jnp.where(adv, wptr, wcur)
                cur = jnp.where(adv, neww, cur)
                wptr = wptr + adv.astype(jnp.int32)
                emit = active & (cur != 0)
                low = cur & (0 - cur)
                fbits = lax.bitcast_convert_type(low.astype(jnp.float32),
                                                jnp.int32)
                pos = lax.shift_right_logical(fbits, 23) - 127
                pidx = (wcur << 4) + pos
                first = jnp.where(emit & (cnt == 0), pidx, first)
                slot = (jnp.minimum(cnt, 31) << 7) + qlane
                plsc.store_scatter(out_v, [slot], pidx, mask=emit)
                cur = jnp.where(emit, cur & (cur - 1), cur)
                cnt = cnt + emit.astype(jnp.int32)
                return wptr, wcur, cur, cnt, first

            init = (zero, zero, zero, zero, zero)
            _, _, _, cnt, first = lax.fori_loop(0, WPQ + NSAMPLE, step, init)

            def pad(s, _):
                slot = (s << 7) + qlane
                plsc.store_scatter(out_v, [slot], first, mask=s >= cnt)
                return 0

            lax.fori_loop(0, NSAMPLE, pad, 0)
            return 0

        lax.fori_loop(0, 8, group_body, 0)
        pltpu.sync_copy(out_v, idx_hbm.at[pl.ds(q0 * 32, 4096)])
        return 0

    lax.fori_loop(0, nch, chunk_body, 0)


def _select(pk):
    nq_total = pk.shape[0]
    pk = pk.reshape(nq_total * WPQ)
    mesh = plsc.VectorSubcoreMesh(core_axis_name="c", subcore_axis_name="s")
    return pl.kernel(
        functools.partial(_select_kernel_body, nq_total=nq_total),
        out_type=jax.ShapeDtypeStruct((nq_total * NSAMPLE,), jnp.int32),
        mesh=mesh,
        compiler_params=pltpu.CompilerParams(needs_layout_passes=False),
        scratch_types=[
            pltpu.VMEM((128 * WPQ,), jnp.int32),
            pltpu.VMEM((128 * NSAMPLE,), jnp.int32),
        ],
    )(pk)


def _gather_kernel_body(feat_hbm, idx_hbm, xyzT_hbm, nqT_hbm, out_hbm,
                        table_v, xt_v, ct_v, idx_v, stage0_v, stage1_v,
                        sem0, sem1, *, B, C, N, npoint):
    wid = lax.axis_index("s") * NC + lax.axis_index("c")
    ncb = C // 8
    tpb = ncb + 1
    ntasks = B * tpb
    QC = 128
    qcn = npoint // QC
    niter = QC * NSAMPLE // 16
    CHW = QC * NSAMPLE
    lanes = lax.broadcasted_iota(jnp.int32, (16,), 0)

    def drain(stage_v, sem, nch):
        for c in range(nch):
            pltpu.make_async_copy(
                stage_v.at[pl.ds(c * NSAMPLE, NSAMPLE), :],
                out_hbm.at[0, c, :, pl.ds(0, QC)], sem).wait()

    def task_body(t, _):
        @pl.when(t < ntasks)
        def _():
            b = t // tpb
            k = t % tpb

            @pl.when(k < ncb)
            def _feature_task():
                pltpu.sync_copy(feat_hbm.at[(b * C + k * 8) // 8],
                                table_v)

                def qchunk(qc, _):
                    qbase = b * npoint + qc * QC
                    pltpu.sync_copy(
                        idx_hbm.at[pl.ds(qbase * NSAMPLE, CHW)], idx_v)

                    def fill_and_send(stage_v, sem):
                        def gstep(i, _):
                            iv = idx_v[pl.ds(i * 16, 16)]
                            sr = lax.shift_right_logical(i, 3)
                            col = (i % 8) * 16
                            for c in range(8):
                                g = plsc.load_gather(
                                    table_v,
                                    [jnp.full((16,), c, jnp.int32), iv])
                                stage_v[c * NSAMPLE + sr, pl.ds(col, 16)] = g
                            return 0

                        lax.fori_loop(0, niter, gstep, 0)
                        for c in range(8):
                            pltpu.async_copy(
                                stage_v.at[pl.ds(c * NSAMPLE, NSAMPLE), :],
                                out_hbm.at[b, k * 8 + c, :,
                                           pl.ds(qc * QC, QC)], sem)

                    @pl.when(qc >= 2)
                    def _():
                        @pl.when(qc % 2 == 0)
                        def _():
                            drain(stage0_v, sem0, 8)

                        @pl.when(qc % 2 == 1)
                        def _():
                            drain(stage1_v, sem1, 8)

                    @pl.when(qc % 2 == 0)
                    def _():
                        fill_and_send(stage0_v, sem0)

                    @pl.when(qc % 2 == 1)
                    def _():
                        fill_and_send(stage1_v, sem1)

                    return 0

                lax.fori_loop(0, qcn, qchunk, 0)
                drain(stage0_v, sem0, 8)
                drain(stage1_v, sem1, 8)

            @pl.when(k == ncb)
            def _xyz_task():
                for d in range(3):
                    pltpu.sync_copy(xyzT_hbm.at[pl.ds((b * 3 + d) * N, N)],
                                    xt_v.at[pl.ds(d * N, N)])
                    pltpu.sync_copy(
                        nqT_hbm.at[pl.ds((b * 3 + d) * npoint, npoint)],
                        ct_v.at[pl.ds(d * npoint, npoint)])

                def qchunk(qc, _):
                    qbase = b * npoint + qc * QC
                    pltpu.sync_copy(
                        idx_hbm.at[pl.ds(qbase * NSAMPLE, CHW)], idx_v)

                    def fill_and_send(stage_v, sem):
                        def gstep(i, _):
                            iv = idx_v[pl.ds(i * 16, 16)]
                            sr = lax.shift_right_logical(i, 3)
                            col = (i % 8) * 16
                            for d in range(3):
                                g = plsc.load_gather(
                                    xt_v, [jnp.full((16,), d * N, jnp.int32)
                                           + iv])
                                ctr = ct_v[pl.ds(d * npoint + qc * QC + col,
                                                 16)]
                                stage_v[d * NSAMPLE + sr, pl.ds(col, 16)] = \
                                    g - ctr
                            return 0

                        lax.fori_loop(0, niter, gstep, 0)
                        for d in range(3):
                            pltpu.async_copy(
                                stage_v.at[pl.ds(d * NSAMPLE, NSAMPLE), :],
                                out_hbm.at[b, C + d, :,
                                           pl.ds(qc * QC, QC)], sem)

                    @pl.when(qc >= 2)
                    def _():
                        @pl.when(qc % 2 == 0)
                        def _():
                            drain(stage0_v, sem0, 3)

                        @pl.when(qc % 2 == 1)
                        def _():
                            drain(stage1_v, sem1, 3)

                    @pl.when(qc % 2 == 0)
                    def _():
                        fill_and_send(stage0_v, sem0)

                    @pl.when(qc % 2 == 1)
                    def _():
                        fill_and_send(stage1_v, sem1)

                    return 0

                lax.fori_loop(0, qcn, qchunk, 0)
                drain(stage0_v, sem0, 3)
                drain(stage1_v, sem1, 3)

        return 0

    ntask_rounds = -(-(B * tpb) // NW)
    lax.fori_loop(0, ntask_rounds, lambda r, _: task_body(wid + r * NW, _), 0)


def _gather(features, idx, xyzT, nqT, npoint):
    B, C, N = features.shape
    mesh = plsc.VectorSubcoreMesh(core_axis_name="c", subcore_axis_name="s")
    QC = 128
    return pl.kernel(
        functools.partial(_gather_kernel_body, B=B, C=C, N=N,
                          npoint=npoint),
        out_type=jax.ShapeDtypeStruct((B, C + 3, NSAMPLE, npoint),
                                      jnp.float32),
        mesh=mesh,
        compiler_params=pltpu.CompilerParams(needs_layout_passes=False),
        scratch_types=[
            pltpu.VMEM((8, N), jnp.float32),
            pltpu.VMEM((3 * N,), jnp.float32),
            pltpu.VMEM((3 * npoint,), jnp.float32),
            pltpu.VMEM((QC * NSAMPLE,), jnp.int32),
            pltpu.VMEM((8 * NSAMPLE, QC), jnp.float32),
            pltpu.VMEM((8 * NSAMPLE, QC), jnp.float32),
            pltpu.SemaphoreType.DMA,
            pltpu.SemaphoreType.DMA,
        ],
    )(features.reshape(B * C // 8, 8, N), idx, xyzT.reshape(B * 3 * N),
      nqT.reshape(B * 3 * npoint))


def kernel(xyz, new_xyz, features):
    B, N, _ = xyz.shape
    npoint = new_xyz.shape[1]
    C = features.shape[1]

    pk = _mask_pack(xyz, new_xyz)
    idx = _select(pk.reshape(B * npoint, N // 16))
    xyzT = jnp.transpose(xyz, (0, 2, 1))
    nqT = jnp.transpose(new_xyz, (0, 2, 1))
    out_sq = _gather(features, idx, xyzT, nqT, npoint)
    return jnp.transpose(out_sq, (0, 1, 3, 2))


# speedup vs baseline: 27.1499x; 1.0401x over previous
import functools

import jax
import jax.numpy as jnp
from jax import lax
from jax.experimental import pallas as pl
from jax.experimental.pallas import tpu as pltpu
from jax.experimental.pallas import tpu_sc as plsc

RADIUS2 = 0.25 * 0.25
NSAMPLE = 32
WPQ = 256
QT = 128

NC, NS = 2, 16
NW = NC * NS


def _mask_pack_block(xyzT_ref, q_ref, pk_ref, *, N):
    pT = xyzT_ref[0]
    q = q_ref[0]
    d2 = None
    for d in range(3):
        dd = q[:, d : d + 1] - pT[d : d + 1, :]
        sq = dd * dd
        d2 = sq if d2 is None else d2 + sq
    wf = (d2 < RADIUS2).astype(jnp.bfloat16)
    i_io = lax.broadcasted_iota(jnp.int32, (N, N // 16), 0)
    g_io = lax.broadcasted_iota(jnp.int32, (N, N // 16), 1)
    pw = lax.shift_left(jnp.int32(1), i_io & 15)
    wt = jnp.where((i_io >> 4) == g_io, pw, 0).astype(jnp.bfloat16)
    pk = lax.dot_general(wf, wt, (((1,), (0,)), ((), ())),
                         preferred_element_type=jnp.float32)
    pk_ref[0] = pk.astype(jnp.int32)


def _mask_pack(xyz, new_xyz):
    B, N, _ = xyz.shape
    npoint = new_xyz.shape[1]
    xyzT = jnp.transpose(xyz, (0, 2, 1))
    return pl.pallas_call(
        functools.partial(_mask_pack_block, N=N),
        grid=(B, npoint // QT),
        in_specs=[
            pl.BlockSpec((1, 3, N), lambda b, qt: (b, 0, 0)),
            pl.BlockSpec((1, QT, 3), lambda b, qt: (b, qt, 0)),
        ],
        out_specs=pl.BlockSpec((1, QT, N // 16), lambda b, qt: (b, qt, 0)),
        out_shape=jax.ShapeDtypeStruct((B, npoint, N // 16), jnp.int32),
    )(xyzT, new_xyz)


def _select_kernel_body(pk_hbm, idx_hbm, pk_v, out_v, *, nq_total):
    wid = lax.axis_index("s") * NC + lax.axis_index("c")
    qpw = nq_total // NW
    nch = qpw // 128
    lanes = lax.broadcasted_iota(jnp.int32, (16,), 0)

    def chunk_body(ch, _):
        q0 = wid * qpw + ch * 128
        pltpu.sync_copy(pk_hbm.at[pl.ds(q0 * WPQ, 128 * WPQ)], pk_v)

        def group_body(g, _):
            qlane = g * 16 + lanes
            zero = jnp.zeros((16,), jnp.int32)

            def step(_, carry):
                wptr, wcur, cur, cnt, first = carry
                active = cnt < 32
                adv = (cur == 0) & (wptr < WPQ) & active
                wclamp = jnp.minimum(wptr, WPQ - 1)
                neww = plsc.load_gather(pk_v, [qlane * WPQ + wclamp])
                wcur = jnp.where(adv, wptr, wcur)
                cur = jnp.where(adv, neww, cur)
                wptr = wptr + adv.astype(jnp.int32)
                emit = active & (cur != 0)
                low = cur & (0 - cur)
                fbits = lax.bitcast_convert_type(low.astype(jnp.float32),
                                                jnp.int32)
                pos = lax.shift_right_logical(fbits, 23) - 127
                pidx = (wcur << 4) + pos
                first = jnp.where(emit & (cnt == 0), pidx, first)
                slot = (jnp.minimum(cnt, 31) << 7) + qlane
                plsc.store_scatter(out_v, [slot], pidx, mask=emit)
                cur = jnp.where(emit, cur & (cur - 1), cur)
                cnt = cnt + emit.astype(jnp.int32)
                return wptr, wcur, cur, cnt, first

            init = (zero, zero, zero, zero, zero)
            _, _, _, cnt, first = lax.fori_loop(0, WPQ + NSAMPLE, step, init)

            def pad(s, _):
                slot = (s << 7) + qlane
                plsc.store_scatter(out_v, [slot], first, mask=s >= cnt)
                return 0

            lax.fori_loop(0, NSAMPLE, pad, 0)
            return 0

        lax.fori_loop(0, 8, group_body, 0)
        pltpu.sync_copy(out_v, idx_hbm.at[pl.ds(q0 * 32, 4096)])
        return 0

    lax.fori_loop(0, nch, chunk_body, 0)


def _select(pk):
    nq_total = pk.shape[0]
    pk = pk.reshape(nq_total * WPQ)
    mesh = plsc.VectorSubcoreMesh(core_axis_name="c", subcore_axis_name="s")
    return pl.kernel(
        functools.partial(_select_kernel_body, nq_total=nq_total),
        out_type=jax.ShapeDtypeStruct((nq_total * NSAMPLE,), jnp.int32),
        mesh=mesh,
        compiler_params=pltpu.CompilerParams(needs_layout_passes=False),
        scratch_types=[
            pltpu.VMEM((128 * WPQ,), jnp.int32),
            pltpu.VMEM((128 * NSAMPLE,), jnp.int32),
        ],
    )(pk)


def _gather_kernel_body(feat_hbm, idx_hbm, xyzT_hbm, nqT_hbm, out_hbm,
                        table_v, xt_v, ct_v, idx0_v, idx1_v,
                        stage0_v, stage1_v, sem0, sem1, isem,
                        *, B, C, N, npoint):
    wid = lax.axis_index("s") * NC + lax.axis_index("c")
    ncb = C // 8
    tpb = ncb + 1
    ntasks = B * tpb
    QC = 128
    qcn = npoint // QC
    niter = QC * NSAMPLE // 16
    CHW = QC * NSAMPLE

    def idx_src(b, qc):
        return idx_hbm.at[pl.ds((b * npoint + qc * QC) * NSAMPLE, CHW)]

    def drain(stage_v, sem, nch):
        for c in range(nch):
            pltpu.make_async_copy(
                stage_v.at[pl.ds(c * NSAMPLE, NSAMPLE), :],
                out_hbm.at[0, c, :, pl.ds(0, QC)], sem).wait()

    def run_task(b, setup, gbody, nch, chbase):
        setup()
        pltpu.async_copy(idx_src(b, 0), idx0_v, isem)

        def fill_and_send(idx_v, stage_v, sem, qc):
            def gstep(ii, _):
                gbody(idx_v, stage_v, qc, ii * 2)
                gbody(idx_v, stage_v, qc, ii * 2 + 1)
                return 0

            lax.fori_loop(0, niter // 2, gstep, 0)
            for c in range(nch):
                pltpu.async_copy(
                    stage_v.at[pl.ds(c * NSAMPLE, NSAMPLE), :],
                    out_hbm.at[b, chbase + c, :,
                               pl.ds(qc * QC, QC)], sem)

        def qchunk(qc, _):
            def phase(idx_v, stage_v, sem):
                pltpu.make_async_copy(idx_src(b, qc), idx_v, isem).wait()

                @pl.when(qc + 1 < qcn)
                def _():
                    pltpu.async_copy(idx_src(b, qc + 1),
                                     idx1_v if idx_v is idx0_v else idx0_v,
                                     isem)

                @pl.when(qc >= 2)
                def _():
                    drain(stage_v, sem, nch)

                fill_and_send(idx_v, stage_v, sem, qc)

            @pl.when(qc % 2 == 0)
            def _():
                phase(idx0_v, stage0_v, sem0)

            @pl.when(qc % 2 == 1)
            def _():
                phase(idx1_v, stage1_v, sem1)

            return 0

        lax.fori_loop(0, qcn, qchunk, 0)
        drain(stage0_v, sem0, nch)
        drain(stage1_v, sem1, nch)

    def task_body(t, _):
        @pl.when(t < ntasks)
        def _():
            b = t // tpb
            k = t % tpb

            @pl.when(k < ncb)
            def _feature_task():
                def setup():
                    pltpu.sync_copy(feat_hbm.at[(b * C + k * 8) // 8],
                                    table_v)

                def gbody(idx_v, stage_v, qc, i):
                    iv = idx_v[pl.ds(i * 16, 16)]
                    sr = lax.shift_right_logical(i, 3)
                    col = (i % 8) * 16
                    for c in range(8):
                        g = plsc.load_gather(
                            table_v, [jnp.full((16,), c, jnp.int32), iv])
                        stage_v[c * NSAMPLE + sr, pl.ds(col, 16)] = g

                run_task(b, setup, gbody, 8, k * 8)

            @pl.when(k == ncb)
            def _xyz_task():
                def setup():
                    for d in range(3):
                        pltpu.sync_copy(
                            xyzT_hbm.at[pl.ds((b * 3 + d) * N, N)],
                            xt_v.at[pl.ds(d * N, N)])
                        pltpu.sync_copy(
                            nqT_hbm.at[pl.ds((b * 3 + d) * npoint, npoint)],
                            ct_v.at[pl.ds(d * npoint, npoint)])

                def gbody(idx_v, stage_v, qc, i):
                    iv = idx_v[pl.ds(i * 16, 16)]
                    sr = lax.shift_right_logical(i, 3)
                    col = (i % 8) * 16
                    for d in range(3):
                        g = plsc.load_gather(
                            xt_v, [jnp.full((16,), d * N, jnp.int32) + iv])
                        ctr = ct_v[pl.ds(d * npoint + qc * QC + col, 16)]
                        stage_v[d * NSAMPLE + sr, pl.ds(col, 16)] = g - ctr

                run_task(b, setup, gbody, 3, C)

        return 0

    ntask_rounds = -(-(B * tpb) // NW)
    lax.fori_loop(0, ntask_rounds, lambda r, _: task_body(wid + r * NW, _), 0)


def _gather(features, idx, xyzT, nqT, npoint):
    B, C, N = features.shape
    mesh = plsc.VectorSubcoreMesh(core_axis_name="c", subcore_axis_name="s")
    QC = 128
    return pl.kernel(
        functools.partial(_gather_kernel_body, B=B, C=C, N=N,
                          npoint=npoint),
        out_type=jax.ShapeDtypeStruct((B, C + 3, NSAMPLE, npoint),
                                      jnp.float32),
        mesh=mesh,
        compiler_params=pltpu.CompilerParams(needs_layout_passes=False),
        scratch_types=[
            pltpu.VMEM((8, N), jnp.float32),
            pltpu.VMEM((3 * N,), jnp.float32),
            pltpu.VMEM((3 * npoint,), jnp.float32),
            pltpu.VMEM((QC * NSAMPLE,), jnp.int32),
            pltpu.VMEM((QC * NSAMPLE,), jnp.int32),
            pltpu.VMEM((8 * NSAMPLE, QC), jnp.float32),
            pltpu.VMEM((8 * NSAMPLE, QC), jnp.float32),
            pltpu.SemaphoreType.DMA,
            pltpu.SemaphoreType.DMA,
            pltpu.SemaphoreType.DMA,
        ],
    )(features.reshape(B * C // 8, 8, N), idx, xyzT.reshape(B * 3 * N),
      nqT.reshape(B * 3 * npoint))


def kernel(xyz, new_xyz, features):
    B, N, _ = xyz.shape
    npoint = new_xyz.shape[1]
    C = features.shape[1]

    pk = _mask_pack(xyz, new_xyz)
    idx = _select(pk.reshape(B * npoint, N // 16))
    xyzT = jnp.transpose(xyz, (0, 2, 1))
    nqT = jnp.transpose(new_xyz, (0, 2, 1))
    out_sq = _gather(features, idx, xyzT, nqT, npoint)
    return jnp.transpose(out_sq, (0, 1, 3, 2))
